# Initial kernel scaffold; baseline (speedup 1.0000x reference)
import jax
import jax.numpy as jnp
from jax.experimental import pallas as pl


def kernel(seq_len, position_ids, cache):
    raise NotImplementedError("write your pallas kernel here")


# speedup vs baseline: 1.0361x; 1.0361x over previous
import functools

import jax
import jax.numpy as jnp
from jax import lax
from jax.experimental import pallas as pl
from jax.experimental.pallas import tpu as pltpu
from jax.experimental.pallas import tpu_sc as plsc

_DIM = 128
_NC, _NS = 2, 16
_NW = _NC * _NS
_IDXW = 128


def _make_gather(n_rows):
    assert n_rows % (_NW * _IDXW) == 0
    chunks_per_w = n_rows // (_NW * _IDXW)
    rows_per_w = chunks_per_w * _IDXW
    mesh = plsc.VectorSubcoreMesh(core_axis_name="c", subcore_axis_name="s")

    @functools.partial(
        pl.kernel,
        mesh=mesh,
        out_type=jax.ShapeDtypeStruct((n_rows, _DIM), jnp.float32),
        scratch_types=[
            pltpu.VMEM((chunks_per_w, _IDXW), jnp.int32),
            pltpu.VMEM((2, _IDXW, _DIM), jnp.float32),
            pltpu.SemaphoreType.DMA,
            pltpu.SemaphoreType.DMA,
        ],
    )
    def gather_kernel(table_hbm, idx_hbm, out_hbm, idx_v, rows_v, gsem, ssem):
        wid = lax.axis_index("s") * _NC + lax.axis_index("c")
        base = wid * rows_per_w
        pltpu.sync_copy(idx_hbm.at[pl.ds(wid * chunks_per_w, chunks_per_w)], idx_v)

        gathers = [None, None]
        scatters = [None, None]
        gathers[0] = pltpu.async_copy(
            table_hbm.at[idx_v.at[0]], rows_v.at[0], gsem)
        for j in range(chunks_per_w):
            cur = j % 2
            nxt = (j + 1) % 2
            if j + 1 < chunks_per_w:
                if scatters[nxt] is not None:
                    scatters[nxt].wait()
                gathers[nxt] = pltpu.async_copy(
                    table_hbm.at[idx_v.at[j + 1]], rows_v.at[nxt], gsem)
            gathers[cur].wait()
            scatters[cur] = pltpu.async_copy(
                rows_v.at[cur], out_hbm.at[pl.ds(base + j * _IDXW, _IDXW)], ssem)
        for s in scatters:
            if s is not None:
                s.wait()

    return gather_kernel


def kernel(seq_len, position_ids, cache):
    b, s = position_ids.shape
    n_rows = b * s
    table = cache.reshape(cache.shape[0], _DIM)
    idx = position_ids.reshape(n_rows // _IDXW, _IDXW)
    out = _make_gather(n_rows)(table, idx)
    return out.reshape(b, s, _DIM // 2, 2)


# speedup vs baseline: 1.0485x; 1.0120x over previous
import functools

import jax
import jax.numpy as jnp
from jax import lax
from jax.experimental import pallas as pl
from jax.experimental.pallas import tpu as pltpu
from jax.experimental.pallas import tpu_sc as plsc

_NC, _NS = 2, 16
_NW = _NC * _NS
_IDXW = 128


def _make_gather(b, s, v, d):
    n_rows = b * s
    assert n_rows % (_NW * _IDXW) == 0
    chunks_per_w = n_rows // (_NW * _IDXW)
    rows_per_w = chunks_per_w * _IDXW
    w_per_b = s // rows_per_w
    mesh = plsc.VectorSubcoreMesh(core_axis_name="c", subcore_axis_name="s")

    @functools.partial(
        pl.kernel,
        mesh=mesh,
        out_type=jax.ShapeDtypeStruct((n_rows, d), jnp.float32),
        scratch_types=[
            pltpu.VMEM((chunks_per_w, _IDXW), jnp.int32),
            pltpu.VMEM((2, _IDXW, d), jnp.float32),
            pltpu.SemaphoreType.DMA,
            pltpu.SemaphoreType.DMA,
            pltpu.SemaphoreType.DMA,
        ],
    )
    def gather_kernel(table_hbm, idx_hbm, out_hbm, idx_v, rows_v, isem, gsem, ssem):
        wid = lax.axis_index("s") * _NC + lax.axis_index("c")
        base = wid * rows_per_w
        bi = wid // w_per_b
        off = (wid % w_per_b) * rows_per_w
        idx_copies = [
            pltpu.async_copy(
                idx_hbm.at[bi, pl.ds(off + k * _IDXW, _IDXW)], idx_v.at[k], isem)
            for k in range(chunks_per_w)
        ]
        for c in idx_copies:
            c.wait()

        gathers = [None, None]
        scatters = [None, None]
        gathers[0] = pltpu.async_copy(
            table_hbm.at[idx_v.at[0]], rows_v.at[0], gsem)
        for j in range(chunks_per_w):
            cur = j % 2
            nxt = (j + 1) % 2
            if j + 1 < chunks_per_w:
                if scatters[nxt] is not None:
                    scatters[nxt].wait()
                gathers[nxt] = pltpu.async_copy(
                    table_hbm.at[idx_v.at[j + 1]], rows_v.at[nxt], gsem)
            gathers[cur].wait()
            scatters[cur] = pltpu.async_copy(
                rows_v.at[cur], out_hbm.at[pl.ds(base + j * _IDXW, _IDXW)], ssem)
        for sc in scatters:
            if sc is not None:
                sc.wait()

    return gather_kernel


def kernel(seq_len, position_ids, cache):
    b, s = position_ids.shape
    v, d2, _ = cache.shape
    d = d2 * 2
    table = cache.reshape(v, d)
    out = _make_gather(b, s, v, d)(table, position_ids)
    return out.reshape(b, s, d2, 2)
